# baseline (speedup 1.0000x reference)
import math

import jax
import jax.numpy as jnp
from jax import lax
from jax.experimental import pallas as pl
from jax.experimental.pallas import tpu as pltpu
from jax.experimental.pallas import tpu_sc as plsc

_NUM_LEVELS = 16
_FEATS = 2
_TABLE = 2 ** 19
_MIN_RES = 16
_MAX_RES = 512
_P1 = 1540863
_P2 = 1256879
_P3 = 1957123
_MASK = _TABLE - 1

_growth = math.exp(math.log(_MAX_RES / _MIN_RES) / (_NUM_LEVELS - 1))
_RES = [int(math.floor(_MIN_RES * _growth ** l + 1e-06)) for l in range(_NUM_LEVELS)]

_NC = 2
_NS = 16
_L = 16
_NW = _NC * _NS

_N = 524288
_PPW = _N // _NW
_C = 512
_NCHUNK = _PPW // _C
_G = _C // _L
_TPL = _TABLE * _FEATS
_SLICE = _TPL // _NS


def _body(xin, tab, out, xyz_v, idx_v, w_v, dst_v, outl_v, spm, sem0, sem1):
    cid = lax.axis_index("c")
    sid = lax.axis_index("s")
    wid = sid * _NC + cid
    lanes = lax.iota(jnp.int32, _L)
    sems = (sem0, sem1)

    def gather_desc(slot, t):
        return pltpu.make_async_copy(
            spm.at[idx_v.at[slot, t]], dst_v.at[slot, t], sems[slot]
        )

    def level_body(lvl, _):
        plsc.subcore_barrier()
        pltpu.sync_copy(
            tab.at[pl.ds(lvl * _TPL + sid * _SLICE, _SLICE)],
            spm.at[pl.ds(sid * _SLICE, _SLICE)],
        )
        plsc.subcore_barrier()

        lvlvec = jnp.zeros((_L,), jnp.int32) + lvl
        resv = jnp.zeros((_L,), jnp.float32)
        for k in range(_NUM_LEVELS):
            resv = jnp.where(lvlvec == k, jnp.float32(_RES[k]), resv)

        def compute_idx(ci, slot):
            base = wid * _PPW + ci * _C
            pltpu.sync_copy(xin.at[pl.ds(base, _C)], xyz_v)

            def idx_body(g, _):
                pb = g * _L
                rows = pb + lanes
                zero = jnp.zeros((_L,), jnp.int32)
                x = plsc.load_gather(xyz_v, [rows, zero])
                y = plsc.load_gather(xyz_v, [rows, zero + 1])
                z = plsc.load_gather(xyz_v, [rows, zero + 2])
                x = jnp.minimum(jnp.maximum(x, 0.0), 1.0)
                y = jnp.minimum(jnp.maximum(y, 0.0), 1.0)
                z = jnp.minimum(jnp.maximum(z, 0.0), 1.0)
                px = x * resv
                py = y * resv
                pz = z * resv
                ix = px.astype(jnp.int32)
                iy = py.astype(jnp.int32)
                iz = pz.astype(jnp.int32)
                fx = px - ix.astype(jnp.float32)
                fy = py - iy.astype(jnp.float32)
                fz = pz - iz.astype(jnp.float32)
                hx = (ix * _P1, ix * _P1 + _P1)
                hy = (iy * _P2, iy * _P2 + _P2)
                hz = (iz * _P3, iz * _P3 + _P3)
                wx = (1.0 - fx, fx)
                wy = (1.0 - fy, fy)
                wz = (1.0 - fz, fz)
                for c in range(8):
                    ox, oy, oz = (c >> 2) & 1, (c >> 1) & 1, c & 1
                    h = jnp.bitwise_xor(jnp.bitwise_xor(hx[ox], hy[oy]), hz[oz])
                    e0 = jnp.bitwise_and(h, _MASK) * 2
                    idx_v[slot, 2 * c, pl.ds(pb, _L)] = e0
                    idx_v[slot, 2 * c + 1, pl.ds(pb, _L)] = e0 + 1
                    w_v[slot, c, pl.ds(pb, _L)] = (wx[ox] * wy[oy]) * wz[oz]
                return _

            lax.fori_loop(0, _G, idx_body, None)

        def fire(slot):
            def fire_body(t, _):
                gather_desc(slot, t).start()
                return _

            lax.fori_loop(0, 2 * 8, fire_body, None)

        def drain(slot):
            def drain_body(t, _):
                gather_desc(slot, t).wait()
                return _

            lax.fori_loop(0, 2 * 8, drain_body, None)

        def combine(ci, slot):
            def comb_body(g, _):
                pb = g * _L
                acc0 = jnp.zeros((_L,), jnp.float32)
                acc1 = jnp.zeros((_L,), jnp.float32)
                for c in range(8):
                    w = w_v[slot, c, pl.ds(pb, _L)]
                    e0 = dst_v[slot, 2 * c, pl.ds(pb, _L)]
                    e1 = dst_v[slot, 2 * c + 1, pl.ds(pb, _L)]
                    acc0 = acc0 + w * e0
                    acc1 = acc1 + w * e1
                outl_v[0, pl.ds(pb, _L)] = acc0
                outl_v[1, pl.ds(pb, _L)] = acc1
                return _

            lax.fori_loop(0, _G, comb_body, None)
            pltpu.sync_copy(
                outl_v,
                out.at[pl.ds(lvl * 2, 2), pl.ds(wid * _PPW + ci * _C, _C)],
            )

        compute_idx(0, 0)
        fire(0)

        def pair_body(i, _):
            c0 = 2 * i
            compute_idx(c0 + 1, 1)
            drain(0)
            fire(1)
            combine(c0, 0)
            compute_idx(c0 + 2, 0)
            drain(1)
            fire(0)
            combine(c0 + 1, 1)
            return _

        lax.fori_loop(0, _NCHUNK // 2 - 1, pair_body, None)
        compute_idx(_NCHUNK - 1, 1)
        drain(0)
        fire(1)
        combine(_NCHUNK - 2, 0)
        drain(1)
        combine(_NCHUNK - 1, 1)

        return _

    lax.fori_loop(0, _NUM_LEVELS, level_body, None)


def _transpose_tc(y):
    bt = 4096

    def tbody(src_ref, dst_ref):
        dst_ref[...] = src_ref[...].T

    return pl.pallas_call(
        tbody,
        grid=(_N // bt,),
        in_specs=[pl.BlockSpec((_NUM_LEVELS * _FEATS, bt), lambda i: (0, i))],
        out_specs=pl.BlockSpec((bt, _NUM_LEVELS * _FEATS), lambda i: (i, 0)),
        out_shape=jax.ShapeDtypeStruct((_N, _NUM_LEVELS * _FEATS), jnp.float32),
    )(y)


def kernel(x01, tables):
    tab = tables.reshape(_NUM_LEVELS * _TABLE * _FEATS)
    mesh = plsc.VectorSubcoreMesh(
        core_axis_name="c", subcore_axis_name="s", num_cores=_NC, num_subcores=_NS
    )
    k = pl.kernel(
        _body,
        out_type=jax.ShapeDtypeStruct((_NUM_LEVELS * _FEATS, _N), jnp.float32),
        mesh=mesh,
        compiler_params=pltpu.CompilerParams(
            needs_layout_passes=False, use_tc_tiling_on_sc=False
        ),
        scratch_types=[
            pltpu.VMEM((_C, 3), jnp.float32),
            pltpu.VMEM((2, 2 * 8, _C), jnp.int32),
            pltpu.VMEM((2, 8, _C), jnp.float32),
            pltpu.VMEM((2, 2 * 8, _C), jnp.float32),
            pltpu.VMEM((2, _C), jnp.float32),
            pltpu.VMEM_SHARED((_TPL,), jnp.float32),
            pltpu.SemaphoreType.DMA,
            pltpu.SemaphoreType.DMA,
        ],
    )
    return _transpose_tc(k(x01, tab))


# speedup vs baseline: 6.0498x; 6.0498x over previous
import math

import jax
import jax.numpy as jnp
from jax import lax
from jax.experimental import pallas as pl
from jax.experimental.pallas import tpu as pltpu
from jax.experimental.pallas import tpu_sc as plsc

_NUM_LEVELS = 16
_FEATS = 2
_TABLE = 2 ** 19
_MIN_RES = 16
_MAX_RES = 512
_P1 = 1540863
_P2 = 1256879
_P3 = 1957123
_MASK = _TABLE - 1

_growth = math.exp(math.log(_MAX_RES / _MIN_RES) / (_NUM_LEVELS - 1))
_RES = [int(math.floor(_MIN_RES * _growth ** l + 1e-06)) for l in range(_NUM_LEVELS)]

_NC = 2
_NS = 16
_L = 16
_NW = _NC * _NS

_N = 524288
_PPW = _N // _NW
_C = 512
_NCHUNK = _PPW // _C
_G = _C // _L
_TPL = _TABLE * _FEATS
_SLICE = _TPL // _NS


def _body(xs, ys, zs, tab, out, xyz_v, idx_v, w_v, dst_v, outl_v, spm, sem0, sem1):
    cid = lax.axis_index("c")
    sid = lax.axis_index("s")
    wid = sid * _NC + cid
    lanes = lax.iota(jnp.int32, _L)
    sems = (sem0, sem1)

    def gather_desc(slot, t):
        return pltpu.make_async_copy(
            spm.at[idx_v.at[slot, t]], dst_v.at[slot, t], sems[slot]
        )

    def level_body(lvl, _):
        plsc.subcore_barrier()
        pltpu.sync_copy(
            tab.at[lvl, pl.ds(sid * _SLICE, _SLICE)],
            spm.at[pl.ds(sid * _SLICE, _SLICE)],
        )
        plsc.subcore_barrier()

        lvlvec = jnp.zeros((_L,), jnp.int32) + lvl
        resv = jnp.zeros((_L,), jnp.float32)
        for k in range(_NUM_LEVELS):
            resv = jnp.where(lvlvec == k, jnp.float32(_RES[k]), resv)

        def compute_idx(ci, slot):
            base = wid * _PPW + ci * _C
            pltpu.sync_copy(xs.at[pl.ds(base, _C)], xyz_v.at[0])
            pltpu.sync_copy(ys.at[pl.ds(base, _C)], xyz_v.at[1])
            pltpu.sync_copy(zs.at[pl.ds(base, _C)], xyz_v.at[2])

            def idx_body(g, _):
                pb = g * _L
                x = xyz_v[0, pl.ds(pb, _L)]
                y = xyz_v[1, pl.ds(pb, _L)]
                z = xyz_v[2, pl.ds(pb, _L)]
                x = jnp.minimum(jnp.maximum(x, 0.0), 1.0)
                y = jnp.minimum(jnp.maximum(y, 0.0), 1.0)
                z = jnp.minimum(jnp.maximum(z, 0.0), 1.0)
                px = x * resv
                py = y * resv
                pz = z * resv
                ix = px.astype(jnp.int32)
                iy = py.astype(jnp.int32)
                iz = pz.astype(jnp.int32)
                fx = px - ix.astype(jnp.float32)
                fy = py - iy.astype(jnp.float32)
                fz = pz - iz.astype(jnp.float32)
                hx = (ix * _P1, ix * _P1 + _P1)
                hy = (iy * _P2, iy * _P2 + _P2)
                hz = (iz * _P3, iz * _P3 + _P3)
                wx = (1.0 - fx, fx)
                wy = (1.0 - fy, fy)
                wz = (1.0 - fz, fz)
                for c in range(8):
                    ox, oy, oz = (c >> 2) & 1, (c >> 1) & 1, c & 1
                    h = jnp.bitwise_xor(jnp.bitwise_xor(hx[ox], hy[oy]), hz[oz])
                    e0 = jnp.bitwise_and(h, _MASK) * 2
                    idx_v[slot, 2 * c, pl.ds(pb, _L)] = e0
                    idx_v[slot, 2 * c + 1, pl.ds(pb, _L)] = e0 + 1
                    w_v[slot, c, pl.ds(pb, _L)] = (wx[ox] * wy[oy]) * wz[oz]
                return _

            lax.fori_loop(0, _G, idx_body, None)

        def fire(slot):
            def fire_body(t, _):
                gather_desc(slot, t).start()
                return _

            lax.fori_loop(0, 2 * 8, fire_body, None)

        def drain(slot):
            def drain_body(t, _):
                gather_desc(slot, t).wait()
                return _

            lax.fori_loop(0, 2 * 8, drain_body, None)

        def combine(ci, slot):
            def comb_body(g, _):
                pb = g * _L
                acc0 = jnp.zeros((_L,), jnp.float32)
                acc1 = jnp.zeros((_L,), jnp.float32)
                for c in range(8):
                    w = w_v[slot, c, pl.ds(pb, _L)]
                    e0 = dst_v[slot, 2 * c, pl.ds(pb, _L)]
                    e1 = dst_v[slot, 2 * c + 1, pl.ds(pb, _L)]
                    acc0 = acc0 + w * e0
                    acc1 = acc1 + w * e1
                outl_v[0, pl.ds(pb, _L)] = acc0
                outl_v[1, pl.ds(pb, _L)] = acc1
                return _

            lax.fori_loop(0, _G, comb_body, None)
            pltpu.sync_copy(
                outl_v,
                out.at[pl.ds(lvl * 2, 2), pl.ds(wid * _PPW + ci * _C, _C)],
            )

        compute_idx(0, 0)
        fire(0)

        def pair_body(i, _):
            c0 = 2 * i
            compute_idx(c0 + 1, 1)
            drain(0)
            fire(1)
            combine(c0, 0)
            compute_idx(c0 + 2, 0)
            drain(1)
            fire(0)
            combine(c0 + 1, 1)
            return _

        lax.fori_loop(0, _NCHUNK // 2 - 1, pair_body, None)
        compute_idx(_NCHUNK - 1, 1)
        drain(0)
        fire(1)
        combine(_NCHUNK - 2, 0)
        drain(1)
        combine(_NCHUNK - 1, 1)

        return _

    lax.fori_loop(0, _NUM_LEVELS, level_body, None)


def _transpose_tc(y):
    bt = 4096

    def tbody(src_ref, dst_ref):
        dst_ref[...] = src_ref[...].T

    return pl.pallas_call(
        tbody,
        grid=(_N // bt,),
        in_specs=[pl.BlockSpec((_NUM_LEVELS * _FEATS, bt), lambda i: (0, i))],
        out_specs=pl.BlockSpec((bt, _NUM_LEVELS * _FEATS), lambda i: (i, 0)),
        out_shape=jax.ShapeDtypeStruct((_N, _NUM_LEVELS * _FEATS), jnp.float32),
    )(y)


def kernel(x01, tables):
    tab = tables.reshape(_NUM_LEVELS, _TABLE * _FEATS)
    mesh = plsc.VectorSubcoreMesh(
        core_axis_name="c", subcore_axis_name="s", num_cores=_NC, num_subcores=_NS
    )
    k = pl.kernel(
        _body,
        out_type=jax.ShapeDtypeStruct((_NUM_LEVELS * _FEATS, _N), jnp.float32),
        mesh=mesh,
        compiler_params=pltpu.CompilerParams(
            needs_layout_passes=False, use_tc_tiling_on_sc=False
        ),
        scratch_types=[
            pltpu.VMEM((3, _C), jnp.float32),
            pltpu.VMEM((2, 2 * 8, _C), jnp.int32),
            pltpu.VMEM((2, 8, _C), jnp.float32),
            pltpu.VMEM((2, 2 * 8, _C), jnp.float32),
            pltpu.VMEM((2, _C), jnp.float32),
            pltpu.VMEM_SHARED((_TPL,), jnp.float32),
            pltpu.SemaphoreType.DMA,
            pltpu.SemaphoreType.DMA,
        ],
    )
    return _transpose_tc(k(x01[:, 0], x01[:, 1], x01[:, 2], tab))


# speedup vs baseline: 6.1412x; 1.0151x over previous
import math

import jax
import jax.numpy as jnp
from jax import lax
from jax.experimental import pallas as pl
from jax.experimental.pallas import tpu as pltpu
from jax.experimental.pallas import tpu_sc as plsc

_NUM_LEVELS = 16
_FEATS = 2
_TABLE = 2 ** 19
_MIN_RES = 16
_MAX_RES = 512
_P1 = 1540863
_P2 = 1256879
_P3 = 1957123
_MASK = _TABLE - 1

_growth = math.exp(math.log(_MAX_RES / _MIN_RES) / (_NUM_LEVELS - 1))
_RES = [int(math.floor(_MIN_RES * _growth ** l + 1e-06)) for l in range(_NUM_LEVELS)]

_NC = 2
_NS = 16
_L = 16
_NW = _NC * _NS

_N = 524288
_PPW = _N // _NW
_C = 512
_NCHUNK = _PPW // _C
_G = _C // _L
_TPL = _TABLE * _FEATS
_SLICE = _TPL // _NS


def _body(xs, ys, zs, tab, out, xyz_v, idx_v, w_v, dst_v, outl_v, spm, sem0, sem1):
    cid = lax.axis_index("c")
    sid = lax.axis_index("s")
    wid = sid * _NC + cid
    lanes = lax.iota(jnp.int32, _L)
    sems = (sem0, sem1)

    def gather_desc(slot, c):
        return pltpu.make_async_copy(
            spm.at[idx_v.at[slot, c]], dst_v.at[slot, c], sems[slot]
        )

    def level_body(lvl, _):
        plsc.subcore_barrier()
        pltpu.sync_copy(
            tab.at[lvl, pl.ds(sid * _SLICE, _SLICE)],
            spm.at[pl.ds(sid * _SLICE, _SLICE)],
        )
        plsc.subcore_barrier()

        lvlvec = jnp.zeros((_L,), jnp.int32) + lvl
        resv = jnp.zeros((_L,), jnp.float32)
        for k in range(_NUM_LEVELS):
            resv = jnp.where(lvlvec == k, jnp.float32(_RES[k]), resv)

        def compute_idx(ci, slot):
            base = wid * _PPW + ci * _C
            pltpu.sync_copy(xs.at[pl.ds(base, _C)], xyz_v.at[0])
            pltpu.sync_copy(ys.at[pl.ds(base, _C)], xyz_v.at[1])
            pltpu.sync_copy(zs.at[pl.ds(base, _C)], xyz_v.at[2])

            def idx_body(g, _):
                pb = g * _L
                x = xyz_v[0, pl.ds(pb, _L)]
                y = xyz_v[1, pl.ds(pb, _L)]
                z = xyz_v[2, pl.ds(pb, _L)]
                px = x * resv
                py = y * resv
                pz = z * resv
                ix = px.astype(jnp.int32)
                iy = py.astype(jnp.int32)
                iz = pz.astype(jnp.int32)
                fx = px - ix.astype(jnp.float32)
                fy = py - iy.astype(jnp.float32)
                fz = pz - iz.astype(jnp.float32)
                hx0 = ix * _P1
                hy0 = iy * _P2
                hz = (iz * _P3, iz * _P3 + _P3)
                hxy = (
                    jnp.bitwise_xor(hx0, hy0),
                    jnp.bitwise_xor(hx0, hy0 + _P2),
                    jnp.bitwise_xor(hx0 + _P1, hy0),
                    jnp.bitwise_xor(hx0 + _P1, hy0 + _P2),
                )
                wx = (1.0 - fx, fx)
                wy = (1.0 - fy, fy)
                wz = (1.0 - fz, fz)
                wxy = (wx[0] * wy[0], wx[0] * wy[1], wx[1] * wy[0], wx[1] * wy[1])
                for c in range(8):
                    oxy, oz = c >> 1, c & 1
                    h = jnp.bitwise_xor(hxy[oxy], hz[oz])
                    e0 = jnp.bitwise_and(h, _MASK) * 2
                    idx_v[slot, oxy * 2 + oz, pl.ds(pb, _L)] = e0
                    idx_v[slot, oxy * 2 + oz, pl.ds(_C + pb, _L)] = e0 + 1
                    w_v[slot, oxy * 2 + oz, pl.ds(pb, _L)] = wxy[oxy] * wz[oz]
                return _

            lax.fori_loop(0, _G, idx_body, None)

        def fire(slot):
            def fire_body(t, _):
                gather_desc(slot, t).start()
                return _

            lax.fori_loop(0, 8, fire_body, None)

        def drain(slot):
            def drain_body(t, _):
                gather_desc(slot, t).wait()
                return _

            lax.fori_loop(0, 8, drain_body, None)

        def combine(ci, slot):
            def comb_body(g, _):
                pb = g * _L
                acc0 = jnp.zeros((_L,), jnp.float32)
                acc1 = jnp.zeros((_L,), jnp.float32)
                for c in range(8):
                    w = w_v[slot, c, pl.ds(pb, _L)]
                    e0 = dst_v[slot, c, pl.ds(pb, _L)]
                    e1 = dst_v[slot, c, pl.ds(_C + pb, _L)]
                    acc0 = acc0 + w * e0
                    acc1 = acc1 + w * e1
                outl_v[0, pl.ds(pb, _L)] = acc0
                outl_v[1, pl.ds(pb, _L)] = acc1
                return _

            lax.fori_loop(0, _G, comb_body, None)
            pltpu.sync_copy(
                outl_v,
                out.at[pl.ds(lvl * 2, 2), pl.ds(wid * _PPW + ci * _C, _C)],
            )

        compute_idx(0, 0)
        fire(0)

        def pair_body(i, _):
            c0 = 2 * i
            compute_idx(c0 + 1, 1)
            drain(0)
            fire(1)
            combine(c0, 0)
            compute_idx(c0 + 2, 0)
            drain(1)
            fire(0)
            combine(c0 + 1, 1)
            return _

        lax.fori_loop(0, _NCHUNK // 2 - 1, pair_body, None)
        compute_idx(_NCHUNK - 1, 1)
        drain(0)
        fire(1)
        combine(_NCHUNK - 2, 0)
        drain(1)
        combine(_NCHUNK - 1, 1)

        return _

    lax.fori_loop(0, _NUM_LEVELS, level_body, None)


def _transpose_tc(y):
    bt = 4096

    def tbody(src_ref, dst_ref):
        dst_ref[...] = src_ref[...].T

    return pl.pallas_call(
        tbody,
        grid=(_N // bt,),
        in_specs=[pl.BlockSpec((_NUM_LEVELS * _FEATS, bt), lambda i: (0, i))],
        out_specs=pl.BlockSpec((bt, _NUM_LEVELS * _FEATS), lambda i: (i, 0)),
        out_shape=jax.ShapeDtypeStruct((_N, _NUM_LEVELS * _FEATS), jnp.float32),
    )(y)


def kernel(x01, tables):
    tab = tables.reshape(_NUM_LEVELS, _TABLE * _FEATS)
    mesh = plsc.VectorSubcoreMesh(
        core_axis_name="c", subcore_axis_name="s", num_cores=_NC, num_subcores=_NS
    )
    k = pl.kernel(
        _body,
        out_type=jax.ShapeDtypeStruct((_NUM_LEVELS * _FEATS, _N), jnp.float32),
        mesh=mesh,
        compiler_params=pltpu.CompilerParams(
            needs_layout_passes=False, use_tc_tiling_on_sc=False
        ),
        scratch_types=[
            pltpu.VMEM((3, _C), jnp.float32),
            pltpu.VMEM((2, 8, 2 * _C), jnp.int32),
            pltpu.VMEM((2, 8, _C), jnp.float32),
            pltpu.VMEM((2, 8, 2 * _C), jnp.float32),
            pltpu.VMEM((2, _C), jnp.float32),
            pltpu.VMEM_SHARED((_TPL,), jnp.float32),
            pltpu.SemaphoreType.DMA,
            pltpu.SemaphoreType.DMA,
        ],
    )
    return _transpose_tc(k(x01[:, 0], x01[:, 1], x01[:, 2], tab))


# speedup vs baseline: 6.3512x; 1.0342x over previous
import math

import jax
import jax.numpy as jnp
from jax import lax
from jax.experimental import pallas as pl
from jax.experimental.pallas import tpu as pltpu
from jax.experimental.pallas import tpu_sc as plsc

_NUM_LEVELS = 16
_FEATS = 2
_TABLE = 2 ** 19
_MIN_RES = 16
_MAX_RES = 512
_P1 = 1540863
_P2 = 1256879
_P3 = 1957123
_MASK = _TABLE - 1

_growth = math.exp(math.log(_MAX_RES / _MIN_RES) / (_NUM_LEVELS - 1))
_RES = [int(math.floor(_MIN_RES * _growth ** l + 1e-06)) for l in range(_NUM_LEVELS)]

_NC = 2
_NS = 16
_L = 16
_NW = _NC * _NS

_N = 524288
_PPW = _N // _NW
_C = 512
_NCHUNK = _PPW // _C
_G = _C // _L
_TPL = _TABLE * _FEATS
_SLICE = _TPL // _NS


def _body(xs, ys, zs, tab, out, xyz_v, idx_v, w_v, dst_v, outl_v, spm, sem0, sem1, semx, semo0, semo1):
    cid = lax.axis_index("c")
    sid = lax.axis_index("s")
    wid = sid * _NC + cid
    lanes = lax.iota(jnp.int32, _L)
    sems = (sem0, sem1)
    semos = (semo0, semo1)

    def gather_desc(slot, c):
        return pltpu.make_async_copy(
            spm.at[idx_v.at[slot, c]], dst_v.at[slot, c], sems[slot]
        )

    def level_body(lvl, _):
        plsc.subcore_barrier()
        pltpu.sync_copy(
            tab.at[lvl, pl.ds(sid * _SLICE, _SLICE)],
            spm.at[pl.ds(sid * _SLICE, _SLICE)],
        )
        plsc.subcore_barrier()

        lvlvec = jnp.zeros((_L,), jnp.int32) + lvl
        resv = jnp.zeros((_L,), jnp.float32)
        for k in range(_NUM_LEVELS):
            resv = jnp.where(lvlvec == k, jnp.float32(_RES[k]), resv)

        def compute_idx(ci, slot):
            base = wid * _PPW + ci * _C
            d0 = pltpu.make_async_copy(xs.at[pl.ds(base, _C)], xyz_v.at[0], semx)
            d1 = pltpu.make_async_copy(ys.at[pl.ds(base, _C)], xyz_v.at[1], semx)
            d2 = pltpu.make_async_copy(zs.at[pl.ds(base, _C)], xyz_v.at[2], semx)
            d0.start()
            d1.start()
            d2.start()
            d0.wait()
            d1.wait()
            d2.wait()

            def idx_body(g, _):
                pb = g * _L
                x = xyz_v[0, pl.ds(pb, _L)]
                y = xyz_v[1, pl.ds(pb, _L)]
                z = xyz_v[2, pl.ds(pb, _L)]
                px = x * resv
                py = y * resv
                pz = z * resv
                ix = px.astype(jnp.int32)
                iy = py.astype(jnp.int32)
                iz = pz.astype(jnp.int32)
                fx = px - ix.astype(jnp.float32)
                fy = py - iy.astype(jnp.float32)
                fz = pz - iz.astype(jnp.float32)
                hx0 = ix * _P1
                hy0 = iy * _P2
                hz = (iz * _P3, iz * _P3 + _P3)
                hxy = (
                    jnp.bitwise_xor(hx0, hy0),
                    jnp.bitwise_xor(hx0, hy0 + _P2),
                    jnp.bitwise_xor(hx0 + _P1, hy0),
                    jnp.bitwise_xor(hx0 + _P1, hy0 + _P2),
                )
                wx = (1.0 - fx, fx)
                wy = (1.0 - fy, fy)
                wz = (1.0 - fz, fz)
                wxy = (wx[0] * wy[0], wx[0] * wy[1], wx[1] * wy[0], wx[1] * wy[1])
                for c in range(8):
                    oxy, oz = c >> 1, c & 1
                    h = jnp.bitwise_xor(hxy[oxy], hz[oz])
                    e0 = jnp.bitwise_and(h, _MASK) * 2
                    idx_v[slot, oxy * 2 + oz, pl.ds(pb, _L)] = e0
                    idx_v[slot, oxy * 2 + oz, pl.ds(_C + pb, _L)] = e0 + 1
                    w_v[slot, oxy * 2 + oz, pl.ds(pb, _L)] = wxy[oxy] * wz[oz]
                return _

            lax.fori_loop(0, _G, idx_body, None)

        def fire(slot):
            def fire_body(t, _):
                gather_desc(slot, t).start()
                return _

            lax.fori_loop(0, 8, fire_body, None)

        def drain(slot):
            def drain_body(t, _):
                gather_desc(slot, t).wait()
                return _

            lax.fori_loop(0, 8, drain_body, None)

        def out_desc(slot, ci):
            return pltpu.make_async_copy(
                outl_v.at[slot],
                out.at[pl.ds(lvl * 2, 2), pl.ds(wid * _PPW + ci * _C, _C)],
                semos[slot],
            )

        def combine(ci, slot):
            @pl.when(ci >= 2)
            def _wait_prev():
                out_desc(slot, ci - 2).wait()

            def comb_body(g, _):
                pb = g * _L
                acc0 = jnp.zeros((_L,), jnp.float32)
                acc1 = jnp.zeros((_L,), jnp.float32)
                for c in range(8):
                    w = w_v[slot, c, pl.ds(pb, _L)]
                    e0 = dst_v[slot, c, pl.ds(pb, _L)]
                    e1 = dst_v[slot, c, pl.ds(_C + pb, _L)]
                    acc0 = acc0 + w * e0
                    acc1 = acc1 + w * e1
                outl_v[slot, 0, pl.ds(pb, _L)] = acc0
                outl_v[slot, 1, pl.ds(pb, _L)] = acc1
                return _

            lax.fori_loop(0, _G, comb_body, None)
            out_desc(slot, ci).start()

        compute_idx(0, 0)
        fire(0)

        def pair_body(i, _):
            c0 = 2 * i
            compute_idx(c0 + 1, 1)
            drain(0)
            fire(1)
            combine(c0, 0)
            compute_idx(c0 + 2, 0)
            drain(1)
            fire(0)
            combine(c0 + 1, 1)
            return _

        lax.fori_loop(0, _NCHUNK // 2 - 1, pair_body, None)
        compute_idx(_NCHUNK - 1, 1)
        drain(0)
        fire(1)
        combine(_NCHUNK - 2, 0)
        drain(1)
        combine(_NCHUNK - 1, 1)
        out_desc(0, _NCHUNK - 2).wait()
        out_desc(1, _NCHUNK - 1).wait()

        return _

    lax.fori_loop(0, _NUM_LEVELS, level_body, None)


def _transpose_tc(y):
    bt = 4096

    def tbody(src_ref, dst_ref):
        dst_ref[...] = src_ref[...].T

    return pl.pallas_call(
        tbody,
        grid=(_N // bt,),
        in_specs=[pl.BlockSpec((_NUM_LEVELS * _FEATS, bt), lambda i: (0, i))],
        out_specs=pl.BlockSpec((bt, _NUM_LEVELS * _FEATS), lambda i: (i, 0)),
        out_shape=jax.ShapeDtypeStruct((_N, _NUM_LEVELS * _FEATS), jnp.float32),
    )(y)


def kernel(x01, tables):
    tab = tables.reshape(_NUM_LEVELS, _TABLE * _FEATS)
    mesh = plsc.VectorSubcoreMesh(
        core_axis_name="c", subcore_axis_name="s", num_cores=_NC, num_subcores=_NS
    )
    k = pl.kernel(
        _body,
        out_type=jax.ShapeDtypeStruct((_NUM_LEVELS * _FEATS, _N), jnp.float32),
        mesh=mesh,
        compiler_params=pltpu.CompilerParams(
            needs_layout_passes=False, use_tc_tiling_on_sc=False
        ),
        scratch_types=[
            pltpu.VMEM((3, _C), jnp.float32),
            pltpu.VMEM((2, 8, 2 * _C), jnp.int32),
            pltpu.VMEM((2, 8, _C), jnp.float32),
            pltpu.VMEM((2, 8, 2 * _C), jnp.float32),
            pltpu.VMEM((2, 2, _C), jnp.float32),
            pltpu.VMEM_SHARED((_TPL,), jnp.float32),
            pltpu.SemaphoreType.DMA,
            pltpu.SemaphoreType.DMA,
            pltpu.SemaphoreType.DMA,
            pltpu.SemaphoreType.DMA,
            pltpu.SemaphoreType.DMA,
        ],
    )
    return _transpose_tc(k(x01[:, 0], x01[:, 1], x01[:, 2], tab))
